# baseline (speedup 1.0000x reference)
import functools

import jax
import jax.numpy as jnp
from jax import lax
from jax.experimental import pallas as pl
from jax.experimental.pallas import tpu as pltpu
from jax.experimental.pallas import tpu_sc as plsc

B = 1024
L = 200
E = 128
NC = 2
NS = 16
NW = NC * NS
ROWS_PER_W = B // NW
CH = 5
CL = L // CH
NB = 8
LA = 4
NU = ROWS_PER_W * CH

_mesh = plsc.VectorSubcoreMesh(core_axis_name="c", subcore_axis_name="s")


@functools.partial(
    pl.kernel,
    mesh=_mesh,
    out_type=jax.ShapeDtypeStruct((B, L, E), jnp.float32),
    scratch_types=[
        pltpu.VMEM((ROWS_PER_W, CH, CL), jnp.int32),
        pltpu.VMEM((L, E), jnp.float32),
        pltpu.VMEM((NB, CL, E), jnp.float32),
    ]
    + [pltpu.SemaphoreType.DMA] * (2 * NB + 2),
)
def _emb_kernel(x_hbm, tok_hbm, pos_hbm, out_hbm, idx_v, pos_v, ubuf, *sems):
    gsems = sems[:NB]
    osems = sems[NB:2 * NB]
    isem, psem = sems[2 * NB], sems[2 * NB + 1]
    wid = lax.axis_index("s") * NC + lax.axis_index("c")
    base = wid * ROWS_PER_W

    icp = pltpu.async_copy(x_hbm.at[pl.ds(base, ROWS_PER_W)], idx_v, isem)
    pcp = pltpu.async_copy(pos_hbm, pos_v, psem)
    icp.wait()

    def add_pos(s, c):
        def add_body(r, _):
            for j in range(E // 16):
                sl = pl.ds(j * 16, 16)
                plsc.addupdate(ubuf.at[s, r, sl], pos_v[c * CL + r, sl])
            return 0
        lax.fori_loop(0, CL, add_body, 0)

    gcp = [None] * NB
    ocp = [None] * NB
    for t in range(NU + LA):
        if t < NU:
            s = t % NB
            if ocp[s] is not None:
                ocp[s].wait()
            i, c = divmod(t, CH)
            gcp[s] = pltpu.async_copy(
                tok_hbm.at[idx_v.at[i, c]], ubuf.at[s], gsems[s]
            )
        if t == 0:
            pcp.wait()
        if t >= LA:
            u = t - LA
            ps = u % NB
            i, c = divmod(u, CH)
            gcp[ps].wait()
            add_pos(ps, c)
            ocp[ps] = pltpu.async_copy(
                ubuf.at[ps], out_hbm.at[base + i, pl.ds(c * CL, CL)], osems[ps]
            )
    for s in range(NB):
        if ocp[s] is not None:
            ocp[s].wait()


IB = 8


def _mask_body(xi_ref, xall_ref, o_ref):
    mi = xi_ref[...] != 0
    mj = xall_ref[...] != 0
    both = mi[:, None, :] & mj[None, :, :]
    o_ref[...] = both.astype(jnp.int32)


def kernel(x, token_table, pos_table):
    x_sc = x.reshape(B, CH, CL)
    x_t = x.T
    mask_t = pl.pallas_call(
        _mask_body,
        grid=(L // IB,),
        in_specs=[
            pl.BlockSpec((IB, B), lambda i: (i, 0)),
            pl.BlockSpec((L, B), lambda i: (0, 0)),
        ],
        out_specs=pl.BlockSpec((IB, L, B), lambda i: (i, 0, 0)),
        out_shape=jax.ShapeDtypeStruct((L, L, B), jnp.int32),
    )(x_t, x_t)
    attn_mask = jnp.transpose(mask_t, (2, 0, 1)).reshape(B, 1, L, L)
    out = _emb_kernel(x_sc, token_table, pos_table)
    return out, attn_mask


# speedup vs baseline: 1.0334x; 1.0334x over previous
import functools

import jax
import jax.numpy as jnp
from jax import lax
from jax.experimental import pallas as pl
from jax.experimental.pallas import tpu as pltpu
from jax.experimental.pallas import tpu_sc as plsc

B = 1024
L = 200
E = 128
NC = 2
NS = 16
NW = NC * NS
ROWS_PER_W = B // NW
CH = 2
CL = L // CH
NB = 3

_mesh = plsc.VectorSubcoreMesh(core_axis_name="c", subcore_axis_name="s")


@functools.partial(
    pl.kernel,
    mesh=_mesh,
    out_type=jax.ShapeDtypeStruct((B, L, E), jnp.float32),
    scratch_types=[
        pltpu.VMEM((ROWS_PER_W, CH, CL), jnp.int32),
        pltpu.VMEM((L, E), jnp.float32),
        pltpu.VMEM((NB, L, E), jnp.float32),
    ]
    + [pltpu.SemaphoreType.DMA] * (2 * NB + 2),
)
def _emb_kernel(x_hbm, tok_hbm, pos_hbm, out_hbm, idx_v, pos_v, rows_v, *sems):
    gsems = sems[:NB]
    osems = sems[NB:2 * NB]
    isem, psem = sems[2 * NB], sems[2 * NB + 1]
    wid = lax.axis_index("s") * NC + lax.axis_index("c")
    base = wid * ROWS_PER_W

    icp = pltpu.async_copy(x_hbm.at[pl.ds(base, ROWS_PER_W)], idx_v, isem)
    pcp = pltpu.async_copy(pos_hbm, pos_v, psem)
    icp.wait()

    def add_rows(b):
        def add_body(r, _):
            for j in range(E // 16):
                sl = pl.ds(j * 16, 16)
                plsc.addupdate(rows_v.at[b, r, sl], pos_v[r, sl])
            return 0
        lax.fori_loop(0, L, add_body, 0)

    gcp = [None] * NB
    ocp = [None] * NB
    for i in range(ROWS_PER_W):
        b = i % NB
        if ocp[b] is not None:
            ocp[b].wait()
        gcp[b] = [
            pltpu.async_copy(
                tok_hbm.at[idx_v.at[i, c]],
                rows_v.at[b, pl.ds(c * CL, CL)],
                gsems[b],
            )
            for c in range(CH)
        ]
        if i == 0:
            pcp.wait()
        if i >= 1:
            pb = (i - 1) % NB
            for cp in gcp[pb]:
                cp.wait()
            add_rows(pb)
            ocp[pb] = pltpu.async_copy(
                rows_v.at[pb], out_hbm.at[base + i - 1], osems[pb]
            )
    lb = (ROWS_PER_W - 1) % NB
    for cp in gcp[lb]:
        cp.wait()
    add_rows(lb)
    ocp[lb] = pltpu.async_copy(
        rows_v.at[lb], out_hbm.at[base + ROWS_PER_W - 1], osems[lb]
    )
    for b in range(NB):
        if ocp[b] is not None:
            ocp[b].wait()


IB = 8


def _mask_body(xi_ref, xall_ref, o_ref):
    mi = xi_ref[...] != 0
    mj = xall_ref[...] != 0
    both = mi[:, None, :] & mj[None, :, :]
    o_ref[...] = both.astype(jnp.int32)


def kernel(x, token_table, pos_table):
    x_sc = x.reshape(B, CH, CL)
    x_t = x.T
    mask_t = pl.pallas_call(
        _mask_body,
        grid=(L // IB,),
        in_specs=[
            pl.BlockSpec((IB, B), lambda i: (i, 0)),
            pl.BlockSpec((L, B), lambda i: (0, 0)),
        ],
        out_specs=pl.BlockSpec((IB, L, B), lambda i: (i, 0, 0)),
        out_shape=jax.ShapeDtypeStruct((L, L, B), jnp.int32),
    )(x_t, x_t)
    attn_mask = jnp.transpose(mask_t, (2, 0, 1)).reshape(B, 1, L, L)
    out = _emb_kernel(x_sc, token_table, pos_table)
    return out, attn_mask
